# baseline (speedup 1.0000x reference)
import jax
import jax.numpy as jnp
from jax.experimental import pallas as pl
from jax.experimental.pallas import tpu as pltpu

N = 500
B = 64
F = 60
C1 = 16
C2 = 64
TOPK = 20
INV21 = 1.0 / 21.0
INV_EPS = 1.0 / (1.0 + 1e-5) ** 0.5
BF = jnp.bfloat16


CH = 8
NCH = B // CH


def _mgdn_kernel(x_hbm, emb_ref, w1_ref, b1_ref, gamma1_ref, beta1_ref,
                 w2_ref, b2_ref, gamma2_ref, beta2_ref, bng_ref, bnb_ref,
                 linw_ref, linb_ref, convw_ref, convb_ref,
                 out_ref, mulx_ref, cos_ref, a_ref, xbuf, xsem):
    def x_copy(c):
        return pltpu.make_async_copy(
            x_hbm.at[pl.ds(c * CH, CH)], xbuf.at[c % 2], xsem.at[c % 2])

    x_copy(0).start()

    w = emb_ref[:]
    inv_nrm = jax.lax.rsqrt(jnp.sum(w * w, axis=1, keepdims=True))
    wn = w * inv_nrm
    cos_ref[:] = jax.lax.dot_general(
        wn, wn, (((1,), (1,)), ((), ())),
        preferred_element_type=jnp.float32)
    col = jax.lax.broadcasted_iota(jnp.int32, (N, N), 1)
    row = jax.lax.broadcasted_iota(jnp.int32, (N, N), 0)

    cos = cos_ref[:]
    for _ in range(TOPK):
        m = jnp.max(cos, axis=1, keepdims=True)
        cand = jnp.where(cos == m, col, N)
        amin = jnp.min(cand, axis=1, keepdims=True)
        cos = jnp.where(col == amin, -jnp.inf, cos)
    cos_ref[:] = cos
    picked = cos_ref[:] == -jnp.inf
    a_ref[:] = (jnp.where(picked, INV21, 0.0)
                + jnp.where(row == col, INV21, 0.0))
    a_bf = a_ref[:].astype(BF)

    wcat = jnp.concatenate([w1_ref[:], convw_ref[:].T], axis=1).astype(BF)
    convb = convb_ref[:]
    s1 = gamma1_ref[:] * INV_EPS
    t1 = b1_ref[:] * s1 + beta1_ref[:]
    s1t = jnp.concatenate([s1] * B, axis=1)
    t1t = jnp.concatenate([t1] * B, axis=1)
    s2 = gamma2_ref[:] * INV_EPS
    t2 = b2_ref[:] * s2 + beta2_ref[:]
    sg = bng_ref[:] * INV_EPS
    sb = bnb_ref[:]
    w2 = w2_ref[:].astype(BF)
    linw = linw_ref[:].astype(BF)
    linb = linb_ref[:]

    h1_parts = []
    for c in range(NCH):
        if c + 1 < NCH:
            x_copy(c + 1).start()
        x_copy(c).wait()
        for i in range(CH):
            b = c * CH + i
            xb = xbuf[c % 2, i].astype(BF)
            hc = jnp.dot(xb, wcat, preferred_element_type=jnp.float32)
            h1_parts.append(hc[:, :C1].astype(BF))
            mulx_ref[b * N:(b + 1) * N, :] = hc[:, C1:] + convb

    h1 = jnp.concatenate(h1_parts, axis=1)
    HB = B // 2
    h2_parts = []
    for half in range(2):
        lo1 = half * HB * C1
        ag1 = jnp.dot(a_bf, h1[:, lo1:lo1 + HB * C1],
                      preferred_element_type=jnp.float32)
        y1 = jnp.maximum(ag1 * s1t[:, lo1:lo1 + HB * C1]
                         + t1t[:, lo1:lo1 + HB * C1], 0.0).astype(BF)
        h2_parts.extend(
            jnp.dot(y1[:, j * C1:(j + 1) * C1], w2,
                    preferred_element_type=jnp.float32).astype(BF)
            for j in range(HB))
    h2 = jnp.concatenate(h2_parts, axis=1)

    emb = emb_ref[:]
    for half in range(2):
        lo = half * HB * C2
        ag2 = jnp.dot(a_bf, h2[:, lo:lo + HB * C2],
                      preferred_element_type=jnp.float32)
        for j in range(HB):
            b = half * HB + j
            zb = ag2[:, j * C2:(j + 1) * C2] * s2 + t2
            m = jnp.max(zb, axis=1, keepdims=True)
            e = jnp.exp(zb - m)
            lse = jnp.log(jnp.sum(e, axis=1, keepdims=True)) + m
            o = (zb - lse) * emb
            o = jnp.maximum(o * sg + sb, 0.0).astype(BF)
            ob = jax.lax.dot_general(
                o, linw, (((1,), (1,)), ((), ())),
                preferred_element_type=jnp.float32) + linb
            out_ref[b * N:(b + 1) * N, :] = ob


def kernel(data, phy_edge_index, net_edge_index, mul_edge_index, mul_emb,
           W1, b1, gamma1, beta1, W2, b2, gamma2, beta2,
           bn_g, bn_b, lin_W, lin_b, conv_W, conv_b):
    f32 = jnp.float32
    row = lambda v: v.reshape(1, -1)

    out, mulx = pl.pallas_call(
        _mgdn_kernel,
        in_specs=[pl.BlockSpec(memory_space=pl.ANY)]
        + [pl.BlockSpec(memory_space=pltpu.MemorySpace.VMEM)] * 15,
        out_shape=[jax.ShapeDtypeStruct((N * B, C2), f32),
                   jax.ShapeDtypeStruct((N * B, C2), f32)],
        scratch_shapes=[pltpu.VMEM((N, N), f32), pltpu.VMEM((N, N), f32),
                        pltpu.VMEM((2, CH, N, F), f32),
                        pltpu.SemaphoreType.DMA((2,))],
        compiler_params=pltpu.CompilerParams(
            skip_device_barrier=True,
            disable_bounds_checks=True),
    )(data, mul_emb, W1, row(b1), row(gamma1), row(beta1),
      W2, row(b2), row(gamma2), row(beta2), row(bn_g), row(bn_b),
      lin_W, row(lin_b), conv_W, row(conv_b))

    return out, mulx


# speedup vs baseline: 1.0033x; 1.0033x over previous
import jax
import jax.numpy as jnp
from jax.experimental import pallas as pl
from jax.experimental.pallas import tpu as pltpu

N = 500
B = 64
F = 60
C1 = 16
C2 = 64
TOPK = 20
INV21 = 1.0 / 21.0
INV_EPS = 1.0 / (1.0 + 1e-5) ** 0.5
BF = jnp.bfloat16


CH = 8
NCH = B // CH


def _mgdn_kernel(x_hbm, emb_ref, w1_ref, b1_ref, gamma1_ref, beta1_ref,
                 w2_ref, b2_ref, gamma2_ref, beta2_ref, bng_ref, bnb_ref,
                 linw_ref, linb_ref, convw_ref, convb_ref,
                 out_ref, mulx_ref, cos_ref, a_ref, xbuf, xsem):
    def x_copy(c):
        return pltpu.make_async_copy(
            x_hbm.at[pl.ds(c * CH, CH)], xbuf.at[c % 2], xsem.at[c % 2])

    x_copy(0).start()

    w = emb_ref[:]
    inv_nrm = jax.lax.rsqrt(jnp.sum(w * w, axis=1, keepdims=True))
    wn = w * inv_nrm
    cos_ref[:] = jax.lax.dot_general(
        wn, wn, (((1,), (1,)), ((), ())),
        preferred_element_type=jnp.float32)
    col = jax.lax.broadcasted_iota(jnp.int32, (N, N), 1)
    row = jax.lax.broadcasted_iota(jnp.int32, (N, N), 0)

    cos = cos_ref[:]
    for _ in range(TOPK):
        m = jnp.max(cos, axis=1, keepdims=True)
        cand = jnp.where(cos == m, col, N)
        amin = jnp.min(cand, axis=1, keepdims=True)
        cos = jnp.where(col == amin, -jnp.inf, cos)
    cos_ref[:] = cos
    picked = cos_ref[:] == -jnp.inf
    a_ref[:] = (jnp.where(picked, INV21, 0.0)
                + jnp.where(row == col, INV21, 0.0))
    a_bf = a_ref[:].astype(BF)

    wcat = jnp.concatenate([w1_ref[:], convw_ref[:].T], axis=1).astype(BF)
    convb = convb_ref[:]
    s1 = gamma1_ref[:] * INV_EPS
    t1 = b1_ref[:] * s1 + beta1_ref[:]
    s1t = jnp.concatenate([s1] * B, axis=1)
    t1t = jnp.concatenate([t1] * B, axis=1)
    s2 = gamma2_ref[:] * INV_EPS
    t2 = b2_ref[:] * s2 + beta2_ref[:]
    sg = bng_ref[:] * INV_EPS
    sb = bnb_ref[:]
    w2 = w2_ref[:].astype(BF)
    linw = linw_ref[:].astype(BF)
    linb = linb_ref[:]

    h1_parts = []
    for c in range(NCH):
        if c + 1 < NCH:
            x_copy(c + 1).start()
        x_copy(c).wait()
        for i in range(CH):
            b = c * CH + i
            xb = xbuf[c % 2, i].astype(BF)
            hc = jnp.dot(xb, wcat, preferred_element_type=jnp.float32)
            h1_parts.append(hc[:, :C1].astype(BF))
            mulx_ref[b * N:(b + 1) * N, :] = hc[:, C1:] + convb

    h1 = jnp.concatenate(h1_parts, axis=1)
    HB = B // 2
    h2_parts = []
    for half in range(2):
        lo1 = half * HB * C1
        ag1 = jnp.dot(a_bf, h1[:, lo1:lo1 + HB * C1],
                      preferred_element_type=jnp.float32)
        y1 = jnp.maximum(ag1 * s1t[:, lo1:lo1 + HB * C1]
                         + t1t[:, lo1:lo1 + HB * C1], 0.0).astype(BF)
        h2_parts.extend(
            jnp.dot(y1[:, j * C1:(j + 1) * C1], w2,
                    preferred_element_type=jnp.float32).astype(BF)
            for j in range(HB))
    h2 = jnp.concatenate(h2_parts, axis=1)

    emb = emb_ref[:]
    for half in range(2):
        lo = half * HB * C2
        ag2 = jnp.dot(a_bf, h2[:, lo:lo + HB * C2],
                      preferred_element_type=jnp.float32)
        for j in range(HB):
            b = half * HB + j
            zb = ag2[:, j * C2:(j + 1) * C2] * s2 + t2
            m = jnp.max(zb, axis=1, keepdims=True)
            e = jnp.exp(zb - m)
            lse = jnp.log(jnp.sum(e, axis=1, keepdims=True)) + m
            o = (zb - lse) * emb
            o = jnp.maximum(o * sg + sb, 0.0).astype(BF)
            ob = jax.lax.dot_general(
                o, linw, (((1,), (1,)), ((), ())),
                preferred_element_type=jnp.float32) + linb
            out_ref[b * N:(b + 1) * N, :] = ob


def kernel(data, phy_edge_index, net_edge_index, mul_edge_index, mul_emb,
           W1, b1, gamma1, beta1, W2, b2, gamma2, beta2,
           bn_g, bn_b, lin_W, lin_b, conv_W, conv_b):
    f32 = jnp.float32
    row = lambda v: v.reshape(1, -1)

    out, mulx = pl.pallas_call(
        _mgdn_kernel,
        in_specs=[pl.BlockSpec(memory_space=pl.ANY)]
        + [pl.BlockSpec(memory_space=pltpu.MemorySpace.VMEM)] * 15,
        out_shape=[jax.ShapeDtypeStruct((N * B, C2), f32),
                   jax.ShapeDtypeStruct((N * B, C2), f32)],
        scratch_shapes=[pltpu.VMEM((N, N), f32), pltpu.VMEM((N, N), f32),
                        pltpu.VMEM((2, CH, N, F), f32),
                        pltpu.SemaphoreType.DMA((2,))],
    )(data, mul_emb, W1, row(b1), row(gamma1), row(beta1),
      W2, row(b2), row(gamma2), row(beta2), row(bn_g), row(bn_b),
      lin_W, row(lin_b), conv_W, row(conv_b))

    return out, mulx
